# baseline (speedup 1.0000x reference)
import functools

import jax
import jax.numpy as jnp
from jax import lax
from jax.experimental import pallas as pl
from jax.experimental.pallas import tpu as pltpu
from jax.experimental.pallas import tpu_sc as plsc

_D = 64
_L = 128
_CHUNK = 512
_NW = 32


def _sc_gather(ids2d, table):
    n_idx_rows, l = ids2d.shape
    assert l == _L
    total = n_idx_rows * _L
    per_w = total // _NW
    rows_per_w = per_w // _L
    jpc = _CHUNK // _L
    g_chunks = per_w // _CHUNK

    mesh = plsc.VectorSubcoreMesh(core_axis_name="c", subcore_axis_name="s")

    @functools.partial(
        pl.kernel,
        out_type=jax.ShapeDtypeStruct((total, _D), jnp.float32),
        mesh=mesh,
        scratch_types=[
            pltpu.VMEM((rows_per_w, _L), jnp.int32),
            pltpu.VMEM((_CHUNK, _D), jnp.float32),
            pltpu.SemaphoreType.DMA,
        ],
        compiler_params=pltpu.CompilerParams(use_tc_tiling_on_sc=False),
    )
    def k(ids_hbm, table_hbm, out_hbm, idx_v, rows_v, sem):
        wid = lax.axis_index("s") * 2 + lax.axis_index("c")
        idx_base = wid * rows_per_w
        out_base = wid * per_w
        pltpu.sync_copy(ids_hbm.at[pl.ds(idx_base, rows_per_w)], idx_v)

        def body(g, carry):
            cps = [
                pltpu.async_copy(
                    table_hbm.at[idx_v.at[g * jpc + j]],
                    rows_v.at[pl.ds(j * _L, _L)],
                    sem,
                )
                for j in range(jpc)
            ]
            for cp in cps:
                cp.wait()
            pltpu.sync_copy(
                rows_v, out_hbm.at[pl.ds(out_base + g * _CHUNK, _CHUNK)]
            )
            return carry

        lax.fori_loop(0, g_chunks, body, 0)

    return k(ids2d, table)


def kernel(input_ids, embedding_table):
    b, s = input_ids.shape
    ids2d = input_ids.reshape(-1).astype(jnp.int32).reshape(-1, _L)
    out = _sc_gather(ids2d, embedding_table)
    return out.reshape(b, s, _D)


# speedup vs baseline: 1.0228x; 1.0228x over previous
import functools

import jax
import jax.numpy as jnp
from jax import lax
from jax.experimental import pallas as pl
from jax.experimental.pallas import tpu as pltpu
from jax.experimental.pallas import tpu_sc as plsc

_D = 64
_L = 128
_CHUNK = 512
_NW = 32


def _sc_gather(ids2d, table):
    n_idx_rows, l = ids2d.shape
    assert l == _L
    total = n_idx_rows * _L
    per_w = total // _NW
    rows_per_w = per_w // _L
    jpc = _CHUNK // _L
    n_chunks = per_w // _CHUNK
    assert n_chunks % 2 == 0 and n_chunks >= 4

    mesh = plsc.VectorSubcoreMesh(core_axis_name="c", subcore_axis_name="s")

    @functools.partial(
        pl.kernel,
        out_type=jax.ShapeDtypeStruct((total, _D), jnp.float32),
        mesh=mesh,
        scratch_types=[
            pltpu.VMEM((rows_per_w, _L), jnp.int32),
            pltpu.VMEM((_CHUNK, _D), jnp.float32),
            pltpu.VMEM((_CHUNK, _D), jnp.float32),
            pltpu.SemaphoreType.DMA,
            pltpu.SemaphoreType.DMA,
            pltpu.SemaphoreType.DMA,
            pltpu.SemaphoreType.DMA,
        ],
        compiler_params=pltpu.CompilerParams(use_tc_tiling_on_sc=False),
    )
    def k(ids_hbm, table_hbm, out_hbm, idx_v, rows0, rows1, g0, g1, o0, o1):
        wid = lax.axis_index("s") * 2 + lax.axis_index("c")
        idx_base = wid * rows_per_w
        out_base = wid * per_w
        pltpu.sync_copy(ids_hbm.at[pl.ds(idx_base, rows_per_w)], idx_v)

        rows = (rows0, rows1)
        gsem = (g0, g1)
        osem = (o0, o1)

        def start_gather(c, b):
            for j in range(jpc):
                pltpu.async_copy(
                    table_hbm.at[idx_v.at[c * jpc + j]],
                    rows[b].at[pl.ds(j * _L, _L)],
                    gsem[b],
                )

        def wait_gather(b):
            pltpu.make_async_copy(
                out_hbm.at[pl.ds(0, _CHUNK)], rows[b], gsem[b]
            ).wait()

        def start_out(c, b):
            pltpu.async_copy(
                rows[b], out_hbm.at[pl.ds(out_base + c * _CHUNK, _CHUNK)],
                osem[b],
            )

        def wait_out(b):
            pltpu.make_async_copy(
                out_hbm.at[pl.ds(0, _CHUNK)], rows[b], osem[b]
            ).wait()

        start_gather(0, 0)
        start_gather(1, 1)
        wait_gather(0)
        start_out(0, 0)

        def body(i, carry):
            c_a = 2 * i + 1
            wait_out(0)
            start_gather(c_a + 1, 0)
            wait_gather(1)
            start_out(c_a, 1)
            wait_out(1)
            start_gather(c_a + 2, 1)
            wait_gather(0)
            start_out(c_a + 1, 0)
            return carry

        lax.fori_loop(0, (n_chunks - 2) // 2, body, 0)

        wait_gather(1)
        start_out(n_chunks - 1, 1)
        wait_out(0)
        wait_out(1)

    return k(ids2d, table)


def kernel(input_ids, embedding_table):
    b, s = input_ids.shape
    ids2d = input_ids.reshape(-1).astype(jnp.int32).reshape(-1, _L)
    out = _sc_gather(ids2d, embedding_table)
    return out.reshape(b, s, _D)
